# baseline (speedup 1.0000x reference)
import functools

import jax
import jax.numpy as jnp
from jax import lax
from jax.experimental import pallas as pl
from jax.experimental.pallas import tpu as pltpu
from jax.experimental.pallas import tpu_sc as plsc

_TSC = 256
_TC_BLK = 256


def _tc_add(x_ref, pe_ref, o_ref):
    o_ref[...] = x_ref[...] + pe_ref[...][None]


def kernel(x, pe):
    b, t, d = x.shape
    nc, ns, nl = 2, 16, 16
    nw = nc * ns
    t1 = t - _TSC
    rows_per_w = _TSC // nw

    mesh = plsc.VectorSubcoreMesh(
        core_axis_name="c", subcore_axis_name="s", num_cores=nc, num_subcores=ns
    )

    @functools.partial(
        pl.kernel,
        out_type=jax.ShapeDtypeStruct((b, _TSC, d), jnp.float32),
        mesh=mesh,
        scratch_types=[
            pltpu.VMEM((rows_per_w, d), jnp.float32),
            pltpu.VMEM((b, rows_per_w, d), jnp.float32),
            pltpu.SemaphoreType.DMA,
            pltpu.SemaphoreType.DMA,
        ],
    )
    def sc_fn(x_hbm, pe_hbm, out_hbm, pe_s, xb, in_sem, out_sem):
        wid = lax.axis_index("s") * nc + lax.axis_index("c")
        r0 = wid * rows_per_w

        pltpu.async_copy(pe_hbm.at[pl.ds(t1 + r0, rows_per_w)], pe_s, in_sem)
        for bb in range(b):
            pltpu.async_copy(
                x_hbm.at[bb, pl.ds(t1 + r0, rows_per_w)], xb.at[bb], in_sem
            )
        pltpu.make_async_copy(pe_hbm.at[pl.ds(0, rows_per_w)], pe_s, in_sem).wait()
        for bb in range(b):
            pltpu.make_async_copy(
                x_hbm.at[0, pl.ds(0, rows_per_w)], xb.at[bb], in_sem
            ).wait()

        @plsc.parallel_loop(0, rows_per_w, 1, unroll=2)
        def row_body(r):
            grp = 4
            for g in range(0, d // nl, grp):
                sls = [pl.ds((g + u) * nl, nl) for u in range(grp)]
                vals = [pe_s[r, sls[u]] for u in range(grp)]
                for bb in range(b):
                    for u in range(grp):
                        plsc.addupdate(xb.at[bb, r, sls[u]], vals[u])

        for bb in range(b):
            pltpu.async_copy(xb.at[bb], out_hbm.at[bb, pl.ds(r0, rows_per_w)], out_sem)
        for bb in range(b):
            pltpu.make_async_copy(
                xb.at[0], out_hbm.at[0, pl.ds(0, rows_per_w)], out_sem
            ).wait()

    sc_out = sc_fn(x, pe)

    nt1 = t1 // _TC_BLK
    tc_out = pl.pallas_call(
        _tc_add,
        grid=(nt1, b),
        in_specs=[
            pl.BlockSpec((1, _TC_BLK, d), lambda i, bb: (bb, i, 0)),
            pl.BlockSpec((_TC_BLK, d), lambda i, bb: (i, 0)),
        ],
        out_specs=pl.BlockSpec((1, _TC_BLK, d), lambda i, bb: (bb, i, 0)),
        out_shape=jax.ShapeDtypeStruct((b, t, d), jnp.float32),
    )(x, pe)

    return lax.dynamic_update_slice(tc_out, sc_out, (0, t1, 0))


# speedup vs baseline: 1.0021x; 1.0021x over previous
import functools

import jax
import jax.numpy as jnp
from jax import lax
from jax.experimental import pallas as pl
from jax.experimental.pallas import tpu as pltpu
from jax.experimental.pallas import tpu_sc as plsc

_TSC = 256
_TC_BLK = 256


def _tc_add(x_ref, pe_ref, o_ref):
    o_ref[...] = x_ref[...] + pe_ref[...][None]


def kernel(x, pe):
    b, t, d = x.shape
    nc, ns, nl = 2, 16, 16
    nw = nc * ns
    t1 = t - _TSC
    rows_per_w = _TSC // nw

    mesh = plsc.VectorSubcoreMesh(
        core_axis_name="c", subcore_axis_name="s", num_cores=nc, num_subcores=ns
    )

    @functools.partial(
        pl.kernel,
        out_type=jax.ShapeDtypeStruct((b, _TSC, d), jnp.float32),
        mesh=mesh,
        scratch_types=[
            pltpu.VMEM((rows_per_w, d), jnp.float32),
            pltpu.VMEM((b, rows_per_w, d), jnp.float32),
            pltpu.SemaphoreType.DMA,
            pltpu.SemaphoreType.DMA,
        ],
    )
    def sc_fn(x_hbm, pe_hbm, out_hbm, pe_s, xb, in_sem, out_sem):
        wid = lax.axis_index("s") * nc + lax.axis_index("c")
        r0 = wid * rows_per_w

        pltpu.async_copy(pe_hbm.at[pl.ds(t1 + r0, rows_per_w)], pe_s, in_sem)
        for bb in range(b):
            pltpu.async_copy(
                x_hbm.at[bb, pl.ds(t1 + r0, rows_per_w)], xb.at[bb], in_sem
            )
        pltpu.make_async_copy(pe_hbm.at[pl.ds(0, rows_per_w)], pe_s, in_sem).wait()
        for bb in range(b):
            pltpu.make_async_copy(
                x_hbm.at[0, pl.ds(0, rows_per_w)], xb.at[bb], in_sem
            ).wait()

        @plsc.parallel_loop(0, rows_per_w, 1, unroll=2)
        def row_body(r):
            grp = 4
            for g in range(0, d // nl, grp):
                sls = [pl.ds((g + u) * nl, nl) for u in range(grp)]
                vals = [pe_s[r, sls[u]] for u in range(grp)]
                for bb in range(b):
                    for u in range(grp):
                        plsc.addupdate(xb.at[bb, r, sls[u]], vals[u])

        for bb in range(b):
            pltpu.async_copy(xb.at[bb], out_hbm.at[bb, pl.ds(r0, rows_per_w)], out_sem)
        for bb in range(b):
            pltpu.make_async_copy(
                xb.at[0], out_hbm.at[0, pl.ds(0, rows_per_w)], out_sem
            ).wait()

    nt1 = t1 // _TC_BLK
    tc_out = pl.pallas_call(
        _tc_add,
        grid=(nt1, b),
        in_specs=[
            pl.BlockSpec((1, _TC_BLK, d), lambda i, bb: (bb, i, 0)),
            pl.BlockSpec((_TC_BLK, d), lambda i, bb: (i, 0)),
        ],
        out_specs=pl.BlockSpec((1, _TC_BLK, d), lambda i, bb: (bb, i, 0)),
        out_shape=jax.ShapeDtypeStruct((b, t, d), jnp.float32),
    )(x, pe)

    sc_out = sc_fn(x, pe)

    return lax.dynamic_update_slice(tc_out, sc_out, (0, t1, 0))


# speedup vs baseline: 1.0119x; 1.0098x over previous
import functools

import jax
import jax.numpy as jnp
from jax import lax
from jax.experimental import pallas as pl
from jax.experimental.pallas import tpu as pltpu
from jax.experimental.pallas import tpu_sc as plsc

_TSC = 256
_TC_BLK = 256


def _tc_add(x_ref, pe_ref, o_ref):
    o_ref[...] = x_ref[...] + pe_ref[...][None]


def kernel(x, pe):
    b, t, d = x.shape
    nc, ns, nl = 2, 16, 16
    nw = nc * ns
    t1 = t - _TSC
    rows_per_w = _TSC // nw

    mesh = plsc.VectorSubcoreMesh(
        core_axis_name="c", subcore_axis_name="s", num_cores=nc, num_subcores=ns
    )

    @functools.partial(
        pl.kernel,
        out_type=jax.ShapeDtypeStruct((b, _TSC, d), jnp.float32),
        mesh=mesh,
        scratch_types=[
            pltpu.VMEM((rows_per_w, d), jnp.float32),
            pltpu.VMEM((b, rows_per_w, d), jnp.float32),
            pltpu.SemaphoreType.DMA,
            pltpu.SemaphoreType.DMA,
        ],
    )
    def sc_fn(x_hbm, pe_hbm, out_hbm, pe_s, xb, in_sem, out_sem):
        wid = lax.axis_index("s") * nc + lax.axis_index("c")
        r0 = wid * rows_per_w

        pltpu.async_copy(pe_hbm.at[pl.ds(t1 + r0, rows_per_w)], pe_s, in_sem)
        for bb in range(b):
            pltpu.async_copy(
                x_hbm.at[bb, pl.ds(t1 + r0, rows_per_w)], xb.at[bb], in_sem
            )
        pltpu.make_async_copy(pe_hbm.at[pl.ds(0, rows_per_w)], pe_s, in_sem).wait()
        for bb in range(b):
            pltpu.make_async_copy(
                x_hbm.at[0, pl.ds(0, rows_per_w)], xb.at[bb], in_sem
            ).wait()

        @plsc.parallel_loop(0, rows_per_w, 1, unroll=2)
        def row_body(r):
            grp = 4
            for g in range(0, d // nl, grp):
                sls = [pl.ds((g + u) * nl, nl) for u in range(grp)]
                vals = [pe_s[r, sls[u]] for u in range(grp)]
                for bb in range(b):
                    for u in range(grp):
                        plsc.addupdate(xb.at[bb, r, sls[u]], vals[u])

        for bb in range(b):
            pltpu.async_copy(xb.at[bb], out_hbm.at[bb, pl.ds(r0, rows_per_w)], out_sem)
        for bb in range(b):
            pltpu.make_async_copy(
                xb.at[0], out_hbm.at[0, pl.ds(0, rows_per_w)], out_sem
            ).wait()

    sc_out = sc_fn(x, pe)

    nt1 = t1 // _TC_BLK
    tc_out = pl.pallas_call(
        _tc_add,
        grid=(nt1, b),
        in_specs=[
            pl.BlockSpec((1, _TC_BLK, d), lambda i, bb: (bb, i, 0)),
            pl.BlockSpec((_TC_BLK, d), lambda i, bb: (i, 0)),
        ],
        out_specs=pl.BlockSpec((1, _TC_BLK, d), lambda i, bb: (bb, i, 0)),
        out_shape=jax.ShapeDtypeStruct((b, t, d), jnp.float32),
    )(x, pe)

    return lax.dynamic_update_slice(tc_out, sc_out, (0, t1, 0))


# speedup vs baseline: 1.9514x; 1.9284x over previous
import jax
import jax.numpy as jnp
from jax.experimental import pallas as pl


def _add_body(pe_ref, x_ref, o_ref):
    o_ref[...] = x_ref[...] + pe_ref[...]


def kernel(x, pe):
    b, t, d = x.shape
    bt = 512
    grid = (t // bt, b)
    return pl.pallas_call(
        _add_body,
        grid=grid,
        in_specs=[
            pl.BlockSpec((bt, d), lambda j, i: (j, 0)),
            pl.BlockSpec((1, bt, d), lambda j, i: (i, j, 0)),
        ],
        out_specs=pl.BlockSpec((1, bt, d), lambda j, i: (i, j, 0)),
        out_shape=jax.ShapeDtypeStruct(x.shape, x.dtype),
    )(pe[:t], x)
